# Initial kernel scaffold; baseline (speedup 1.0000x reference)
import jax
import jax.numpy as jnp
from jax.experimental import pallas as pl


def kernel(x, edge_index, initial_weight, pool_W, Wz, Uz, bz, Wr, Ur, br, Wh, Uh, bh):
    raise NotImplementedError("write your pallas kernel here")


# speedup vs baseline: 14.5428x; 14.5428x over previous
import functools

import jax
import jax.numpy as jnp
from jax import lax
from jax.experimental import pallas as pl
from jax.experimental.pallas import tpu as pltpu
from jax.experimental.pallas import tpu_sc as plsc

N = 10000
E = 320000
F = 128
C = 128
T = 3

NC = 2
NS = 16
NW = NC * NS
EPW = E // NW
K = 80
NCHUNK = EPW // K
NPAD = 10112
RPT = NPAD // NS
LANES = 16

_SLOPE = (1.0 / 8.0 + 1.0 / 3.0) / 2.0

def _sc_degree_body(dst_hbm, ones_hbm, zeros_hbm, hist_out,
                    dst_v, ones_v, hist_sh, sem):
    c = lax.axis_index("c")
    s = lax.axis_index("s")
    wid = s * NC + c
    pltpu.sync_copy(zeros_hbm.at[pl.ds(s * RPT, RPT)],
                    hist_sh.at[pl.ds(s * RPT, RPT)])
    pltpu.sync_copy(dst_hbm.at[wid], dst_v)
    pltpu.sync_copy(ones_hbm, ones_v)
    plsc.subcore_barrier()

    def add_body(i, carry):
        pltpu.sync_copy(ones_v, hist_sh.at[dst_v.at[i]], add=True)
        return carry

    lax.fori_loop(0, NCHUNK, add_body, 0)
    plsc.subcore_barrier()
    pltpu.sync_copy(hist_sh.at[pl.ds(s * RPT, RPT)],
                    hist_out.at[c, pl.ds(s * RPT, RPT)])


def _sc_scatter_body(g_hbm, src_hbm, dst_hbm, zeros_hbm, s_out,
                     src_v, dst_v, rows_v, s_sh, sem):
    c = lax.axis_index("c")
    s = lax.axis_index("s")
    wid = s * NC + c
    pltpu.sync_copy(zeros_hbm.at[pl.ds(s * RPT, RPT)],
                    s_sh.at[pl.ds(s * RPT, RPT)])
    pltpu.sync_copy(src_hbm.at[wid], src_v)
    pltpu.sync_copy(dst_hbm.at[wid], dst_v)
    plsc.subcore_barrier()

    def chunk_body(i, carry):
        pltpu.async_copy(g_hbm.at[src_v.at[i]], rows_v, sem).wait()
        pltpu.sync_copy(rows_v, s_sh.at[dst_v.at[i]], add=True)
        return carry

    lax.fori_loop(0, NCHUNK, chunk_body, 0)
    plsc.subcore_barrier()
    pltpu.sync_copy(s_sh.at[pl.ds(s * RPT, RPT)],
                    s_out.at[c, pl.ds(s * RPT, RPT)])


def _wchain_body(iw, wz, uz, bz, wr, ur, br, wh, uh, bh, out):
    w = iw[...]
    for t in range(T):
        z = jax.nn.sigmoid(jnp.dot(wz[...], w, preferred_element_type=jnp.float32)
                           + jnp.dot(uz[...], w, preferred_element_type=jnp.float32)
                           + bz[...])
        r = jax.nn.sigmoid(jnp.dot(wr[...], w, preferred_element_type=jnp.float32)
                           + jnp.dot(ur[...], w, preferred_element_type=jnp.float32)
                           + br[...])
        hc = jnp.tanh(jnp.dot(wh[...], w, preferred_element_type=jnp.float32)
                      + jnp.dot(uh[...], r * w, preferred_element_type=jnp.float32)
                      + bh[...])
        w = (1.0 - z) * w + z * hc
        out[t] = w


NB = 10
BR = N // NB


def _dinv_body(hp_ref, dinv_ref):
    deg = hp_ref[0, :N, 0] + hp_ref[1, :N, 0] + 1.0
    dinv_ref[...] = lax.rsqrt(deg)[:, None]


def _g_body(x_ref, pool_ref, wall_ref, dinv_ref, g_ref):
    xb = x_ref[0]
    h1 = lax.dot_general(xb, pool_ref[...], (((1,), (1,)), ((), ())),
                         preferred_element_type=jnp.float32)
    h2 = jnp.dot(h1, wall_ref[0], preferred_element_type=jnp.float32)
    g_ref[0] = h2 * dinv_ref[...]


def _epi_body(s0_ref, s1_ref, s2_ref, g_ref, dinv_ref, out_ref):
    dinv = dinv_ref[...]
    for t, s_ref in enumerate((s0_ref, s1_ref, s2_ref)):
        acc = s_ref[0] + s_ref[1] + g_ref[t]
        v = dinv * acc
        out_ref[:, t, :] = jnp.where(v >= 0, v, _SLOPE * v)


@functools.lru_cache(maxsize=1)
def _sc_kernels():
    mesh = plsc.VectorSubcoreMesh(core_axis_name="c", subcore_axis_name="s")
    sc_degree = pl.kernel(
        _sc_degree_body,
        out_type=jax.ShapeDtypeStruct((NC, NPAD, LANES), jnp.float32),
        mesh=mesh,
        scratch_types=[
            pltpu.VMEM((NCHUNK, K), jnp.int32),
            pltpu.VMEM((K, LANES), jnp.float32),
            pltpu.VMEM_SHARED((NPAD, LANES), jnp.float32),
            pltpu.SemaphoreType.DMA,
        ],
    )
    sc_scatter = pl.kernel(
        _sc_scatter_body,
        out_type=jax.ShapeDtypeStruct((NC, NPAD, C), jnp.float32),
        mesh=mesh,
        scratch_types=[
            pltpu.VMEM((NCHUNK, K), jnp.int32),
            pltpu.VMEM((NCHUNK, K), jnp.int32),
            pltpu.VMEM((K, C), jnp.float32),
            pltpu.VMEM_SHARED((NPAD, C), jnp.float32),
            pltpu.SemaphoreType.DMA,
        ],
    )
    return sc_degree, sc_scatter


def kernel(x, edge_index, initial_weight, pool_W, Wz, Uz, bz, Wr, Ur, br,
           Wh, Uh, bh):
    _sc_degree, _sc_scatter = _sc_kernels()
    src = edge_index[0].reshape(NW, NCHUNK, K)
    dst = edge_index[1].reshape(NW, NCHUNK, K)
    zeros = jnp.zeros((NPAD, C), jnp.float32)
    ones16 = jnp.ones((K, LANES), jnp.float32)
    zeros16 = jnp.zeros((NPAD, LANES), jnp.float32)

    hp = _sc_degree(dst, ones16, zeros16)

    dinv = pl.pallas_call(
        _dinv_body,
        out_shape=jax.ShapeDtypeStruct((N, 1), jnp.float32),
    )(hp)

    w_all = pl.pallas_call(
        _wchain_body,
        out_shape=jax.ShapeDtypeStruct((T, C, C), jnp.float32),
    )(initial_weight, Wz, Uz, bz, Wr, Ur, br, Wh, Uh, bh)

    g = pl.pallas_call(
        _g_body,
        grid=(T, NB),
        in_specs=[
            pl.BlockSpec((1, BR, F), lambda t, b: (t, b, 0)),
            pl.BlockSpec((C, F), lambda t, b: (0, 0)),
            pl.BlockSpec((1, C, C), lambda t, b: (t, 0, 0)),
            pl.BlockSpec((BR, 1), lambda t, b: (b, 0)),
        ],
        out_specs=pl.BlockSpec((1, BR, C), lambda t, b: (t, b, 0)),
        out_shape=jax.ShapeDtypeStruct((T, N, C), jnp.float32),
    )(x, pool_W, w_all, dinv)

    s_parts = [_sc_scatter(g[t], src, dst, zeros) for t in range(T)]

    out = pl.pallas_call(
        _epi_body,
        grid=(NB,),
        in_specs=[
            pl.BlockSpec((NC, BR, C), lambda b: (0, b, 0)),
            pl.BlockSpec((NC, BR, C), lambda b: (0, b, 0)),
            pl.BlockSpec((NC, BR, C), lambda b: (0, b, 0)),
            pl.BlockSpec((T, BR, C), lambda b: (0, b, 0)),
            pl.BlockSpec((BR, 1), lambda b: (b, 0)),
        ],
        out_specs=pl.BlockSpec((BR, T, C), lambda b: (b, 0, 0)),
        out_shape=jax.ShapeDtypeStruct((N, T, C), jnp.float32),
    )(s_parts[0], s_parts[1], s_parts[2], g, dinv)
    return out
